# Initial kernel scaffold; baseline (speedup 1.0000x reference)
import jax
import jax.numpy as jnp
from jax.experimental import pallas as pl


def kernel(x, edge_index, batch, params):
    raise NotImplementedError("write your pallas kernel here")


# speedup vs baseline: 1.0074x; 1.0074x over previous
import functools
import math

import jax
import jax.numpy as jnp
from jax.experimental import pallas as pl

_N_NODES = 10000
_N_PAD = 10240
_BLK = 256
_N_GRAPHS = 64
_HID = 64
_RATIO_NUM = 11
_RATIO_DEN = 100


def _conv_body(agg_ref, cnt_ref, x_ref, wr_ref, b_ref, wroot_ref, o_ref, *,
               mean, act):
    agg = agg_ref[...]
    if mean:
        cnt = jnp.maximum(cnt_ref[...], 1.0)
        agg = agg / cnt
    y = (jnp.dot(agg, wr_ref[...], preferred_element_type=jnp.float32)
         + b_ref[...]
         + jnp.dot(x_ref[...], wroot_ref[...],
                   preferred_element_type=jnp.float32))
    if act == "relu":
        y = jnp.maximum(y, 0.0)
    o_ref[...] = y


def _conv_transform(agg, cnt, x, w_rel, b_rel, w_root, mean, act):
    n, din = x.shape
    h = w_rel.shape[1]
    grid = (n // _BLK,)
    return pl.pallas_call(
        functools.partial(_conv_body, mean=mean, act=act),
        grid=grid,
        in_specs=[
            pl.BlockSpec((_BLK, din), lambda i: (i, 0)),
            pl.BlockSpec((_BLK, 1), lambda i: (i, 0)),
            pl.BlockSpec((_BLK, din), lambda i: (i, 0)),
            pl.BlockSpec((din, h), lambda i: (0, 0)),
            pl.BlockSpec((1, h), lambda i: (0, 0)),
            pl.BlockSpec((din, h), lambda i: (0, 0)),
        ],
        out_specs=pl.BlockSpec((_BLK, h), lambda i: (i, 0)),
        out_shape=jax.ShapeDtypeStruct((n, h), jnp.float32),
    )(agg, cnt, x, w_rel, b_rel, w_root)


def _pool_body(ohm_ref, hext_ref, o_ref):
    @pl.when(pl.program_id(0) == 0)
    def _init():
        o_ref[...] = jnp.zeros_like(o_ref)

    o_ref[...] += jax.lax.dot_general(
        ohm_ref[...], hext_ref[...],
        (((0,), (0,)), ((), ())),
        preferred_element_type=jnp.float32)


def _global_pool(ohm, hext):
    n = ohm.shape[0]
    return pl.pallas_call(
        _pool_body,
        grid=(n // _BLK,),
        in_specs=[
            pl.BlockSpec((_BLK, _N_GRAPHS), lambda i: (i, 0)),
            pl.BlockSpec((_BLK, 128), lambda i: (i, 0)),
        ],
        out_specs=pl.BlockSpec((_N_GRAPHS, 128), lambda i: (0, 0)),
        out_shape=jax.ShapeDtypeStruct((_N_GRAPHS, 128), jnp.float32),
    )(ohm, hext)


def _head_body(z_ref, w1_ref, b1_ref, w2_ref, b2_ref, o_ref, *, n_classes):
    z = jnp.maximum(
        jnp.dot(z_ref[...], w1_ref[...], preferred_element_type=jnp.float32)
        + b1_ref[...], 0.0)
    logits = (jnp.dot(z, w2_ref[...], preferred_element_type=jnp.float32)
              + b2_ref[...])
    col = jax.lax.broadcasted_iota(jnp.int32, logits.shape, 1)
    valid = col < n_classes
    neg = jnp.float32(-1e30)
    masked = jnp.where(valid, logits, neg)
    m = jnp.max(masked, axis=1, keepdims=True)
    e = jnp.where(valid, jnp.exp(masked - m), 0.0)
    lse = jnp.log(jnp.sum(e, axis=1, keepdims=True))
    o_ref[...] = masked - m - lse


def _mlp_head(z, w1, b1, w2p, b2p, n_classes):
    g, dz = z.shape
    return pl.pallas_call(
        functools.partial(_head_body, n_classes=n_classes),
        in_specs=[
            pl.BlockSpec((g, dz), lambda: (0, 0)),
            pl.BlockSpec((dz, _HID), lambda: (0, 0)),
            pl.BlockSpec((1, _HID), lambda: (0, 0)),
            pl.BlockSpec((_HID, 128), lambda: (0, 0)),
            pl.BlockSpec((1, 128), lambda: (0, 0)),
        ],
        out_specs=pl.BlockSpec((g, 128), lambda: (0, 0)),
        out_shape=jax.ShapeDtypeStruct((g, 128), jnp.float32),
    )(z, w1, b1, w2p, b2p)


def _segment(msg, dst, n):
    return jax.ops.segment_sum(msg, dst, num_segments=n)


def _graph_conv(h, src, dst, valid, pp, mean, act):
    msg = h[src] * valid[:, None]
    agg = _segment(msg, dst, _N_PAD)
    if mean:
        cnt = _segment(valid, dst, _N_PAD)[:, None]
    else:
        cnt = jnp.zeros((_N_PAD, 1), jnp.float32)
    return _conv_transform(agg, cnt, h, pp['W_rel'], pp['b_rel'][None, :],
                           pp['W_root'], mean, act)


def _mean_pool(h, ohm):
    ones = jnp.ones((_N_PAD, 1), jnp.float32)
    pad = jnp.zeros((_N_PAD, 128 - _HID - 1), jnp.float32)
    hext = jnp.concatenate([h, ones, pad], axis=1)
    out = _global_pool(ohm, hext)
    s = out[:, :_HID]
    cnt = jnp.maximum(out[:, _HID:_HID + 1], 1.0)
    return s / cnt


def _pad_pool_params(pp):
    w_rel = jnp.zeros((_HID, 128), jnp.float32).at[:, :1].set(pp['W_rel'])
    w_root = jnp.zeros((_HID, 128), jnp.float32).at[:, :1].set(pp['W_root'])
    b = jnp.zeros((1, 128), jnp.float32).at[0, :1].set(pp['b_rel'])
    return w_rel, b, w_root


def _sag_pool(h, src, dst, valid, batch, node_valid, starts, total, pp):
    w_rel, b, w_root = _pad_pool_params(pp)
    msg = h[src] * valid[:, None]
    agg = _segment(msg, dst, _N_PAD)
    cnt = jnp.zeros((_N_PAD, 1), jnp.float32)
    score = _conv_transform(agg, cnt, h, w_rel, b, w_root, False, "none")[:, 0]

    n = _N_PAD
    nvalid = _segment(node_valid.astype(jnp.int32), batch, _N_GRAPHS)
    k = (nvalid * _RATIO_NUM + (_RATIO_DEN - 1)) // _RATIO_DEN
    sort_key = jnp.where(node_valid > 0, -score, jnp.inf)
    order = jnp.lexsort((sort_key, batch))
    rank_sorted = jnp.arange(n, dtype=jnp.int32) - starts[batch[order]]
    rank = jnp.zeros((n,), jnp.int32).at[order].set(rank_sorted)
    keep = (node_valid > 0) & (rank < k[batch])
    keep_f = keep.astype(jnp.float32)
    new_h = jnp.where(keep[:, None], h * jnp.tanh(score)[:, None], 0.0)
    new_valid = valid * keep_f[src] * keep_f[dst]
    return new_h, new_valid, keep_f


def kernel(x, edge_index, batch, params):
    src = edge_index[0]
    dst = edge_index[1]
    n_pad_rows = _N_PAD - _N_NODES

    xp = jnp.pad(x, ((0, n_pad_rows), (0, 0)))
    batch_p = jnp.concatenate(
        [batch, jnp.full((n_pad_rows,), _N_GRAPHS - 1, jnp.int32)])
    node_valid = jnp.concatenate(
        [jnp.ones((_N_NODES,), jnp.float32),
         jnp.zeros((n_pad_rows,), jnp.float32)])
    valid = jnp.ones((src.shape[0],), jnp.float32)

    ones_i = jnp.ones((_N_PAD,), jnp.int32)
    total = _segment(ones_i, batch_p, _N_GRAPHS)
    starts = jnp.cumsum(total) - total

    oh = (batch_p[:, None] == jnp.arange(_N_GRAPHS)[None, :]).astype(
        jnp.float32)

    n_layers = 6
    h = _graph_conv(xp, src, dst, valid, params['conv1'], True, "relu")
    ohm = oh * node_valid[:, None]
    xs = [_mean_pool(h, ohm)]
    pool_i = 0
    n_convs = n_layers - 1
    for i in range(n_convs):
        h = _graph_conv(h, src, dst, valid, params['convs'][i], True, "relu")
        xs.append(_mean_pool(h, ohm))
        if i % 2 == 0 and i < n_convs - 1:
            h, valid, node_valid = _sag_pool(
                h, src, dst, valid, batch_p, node_valid, starts, total,
                params['pools'][pool_i])
            ohm = oh * node_valid[:, None]
            pool_i += 1

    z = jnp.concatenate(xs, axis=1)
    n_classes = params['lin2_W'].shape[1]
    w2p = jnp.zeros((_HID, 128), jnp.float32).at[:, :n_classes].set(
        params['lin2_W'])
    b2p = jnp.zeros((1, 128), jnp.float32).at[0, :n_classes].set(
        params['lin2_b'])
    out = _mlp_head(z, params['lin1_W'], params['lin1_b'][None, :],
                    w2p, b2p, n_classes)
    return out[:, :n_classes]


# speedup vs baseline: 1.9265x; 1.9123x over previous
import functools
import math

import jax
import jax.numpy as jnp
from jax.experimental import pallas as pl

_N_NODES = 10000
_N_PAD = 10240
_BLK = 256
_N_GRAPHS = 64
_HID = 64
_KTAB = [math.ceil(0.11 * n) for n in range(_N_NODES + 1)]


def _conv_body(agg_ref, cnt_ref, x_ref, wr_ref, b_ref, wroot_ref, o_ref, *,
               mean, act):
    agg = agg_ref[...]
    if mean:
        cnt = jnp.maximum(cnt_ref[...], 1.0)
        agg = agg / cnt
    y = (jnp.dot(agg, wr_ref[...], preferred_element_type=jnp.float32)
         + b_ref[...]
         + jnp.dot(x_ref[...], wroot_ref[...],
                   preferred_element_type=jnp.float32))
    if act == "relu":
        y = jnp.maximum(y, 0.0)
    o_ref[...] = y


def _conv_transform(agg, cnt, x, w_rel, b_rel, w_root, mean, act):
    n, din = x.shape
    h = w_rel.shape[1]
    grid = (n // _BLK,)
    return pl.pallas_call(
        functools.partial(_conv_body, mean=mean, act=act),
        grid=grid,
        in_specs=[
            pl.BlockSpec((_BLK, din), lambda i: (i, 0)),
            pl.BlockSpec((_BLK, 1), lambda i: (i, 0)),
            pl.BlockSpec((_BLK, din), lambda i: (i, 0)),
            pl.BlockSpec((din, h), lambda i: (0, 0)),
            pl.BlockSpec((1, h), lambda i: (0, 0)),
            pl.BlockSpec((din, h), lambda i: (0, 0)),
        ],
        out_specs=pl.BlockSpec((_BLK, h), lambda i: (i, 0)),
        out_shape=jax.ShapeDtypeStruct((n, h), jnp.float32),
    )(agg, cnt, x, w_rel, b_rel, w_root)


def _pool_body(ohm_ref, hext_ref, o_ref):
    @pl.when(pl.program_id(0) == 0)
    def _init():
        o_ref[...] = jnp.zeros_like(o_ref)

    o_ref[...] += jax.lax.dot_general(
        ohm_ref[...], hext_ref[...],
        (((0,), (0,)), ((), ())),
        preferred_element_type=jnp.float32)


def _global_pool(ohm, hext):
    n = ohm.shape[0]
    return pl.pallas_call(
        _pool_body,
        grid=(n // _BLK,),
        in_specs=[
            pl.BlockSpec((_BLK, _N_GRAPHS), lambda i: (i, 0)),
            pl.BlockSpec((_BLK, 128), lambda i: (i, 0)),
        ],
        out_specs=pl.BlockSpec((_N_GRAPHS, 128), lambda i: (0, 0)),
        out_shape=jax.ShapeDtypeStruct((_N_GRAPHS, 128), jnp.float32),
    )(ohm, hext)


def _head_body(z_ref, w1_ref, b1_ref, w2_ref, b2_ref, o_ref, *, n_classes):
    z = jnp.maximum(
        jnp.dot(z_ref[...], w1_ref[...], preferred_element_type=jnp.float32)
        + b1_ref[...], 0.0)
    logits = (jnp.dot(z, w2_ref[...], preferred_element_type=jnp.float32)
              + b2_ref[...])
    col = jax.lax.broadcasted_iota(jnp.int32, logits.shape, 1)
    valid = col < n_classes
    neg = jnp.float32(-1e30)
    masked = jnp.where(valid, logits, neg)
    m = jnp.max(masked, axis=1, keepdims=True)
    e = jnp.where(valid, jnp.exp(masked - m), 0.0)
    lse = jnp.log(jnp.sum(e, axis=1, keepdims=True))
    o_ref[...] = masked - m - lse


def _mlp_head(z, w1, b1, w2p, b2p, n_classes):
    g, dz = z.shape
    return pl.pallas_call(
        functools.partial(_head_body, n_classes=n_classes),
        in_specs=[
            pl.BlockSpec((g, dz), lambda: (0, 0)),
            pl.BlockSpec((dz, _HID), lambda: (0, 0)),
            pl.BlockSpec((1, _HID), lambda: (0, 0)),
            pl.BlockSpec((_HID, 128), lambda: (0, 0)),
            pl.BlockSpec((1, 128), lambda: (0, 0)),
        ],
        out_specs=pl.BlockSpec((g, 128), lambda: (0, 0)),
        out_shape=jax.ShapeDtypeStruct((g, 128), jnp.float32),
    )(z, w1, b1, w2p, b2p)


def _spmm_body(a_ref, h_ref, o_ref):
    @pl.when(pl.program_id(1) == 0)
    def _init():
        o_ref[...] = jnp.zeros_like(o_ref)

    o_ref[...] += jnp.dot(a_ref[...], h_ref[...],
                          preferred_element_type=jnp.float32,
                          precision=jax.lax.Precision.HIGHEST)


def _spmm(a0, hext):
    n = a0.shape[0]
    c = hext.shape[1]
    return pl.pallas_call(
        _spmm_body,
        grid=(n // _BLK, n // _BLK),
        in_specs=[
            pl.BlockSpec((_BLK, _BLK), lambda i, j: (i, j)),
            pl.BlockSpec((_BLK, c), lambda i, j: (j, 0)),
        ],
        out_specs=pl.BlockSpec((_BLK, c), lambda i, j: (i, 0)),
        out_shape=jax.ShapeDtypeStruct((n, c), jnp.float32),
    )(a0, hext)


def _message_pass(a0, h, kmask):
    d = h.shape[1]
    c = ((d + 1 + 127) // 128) * 128
    hext = jnp.zeros((_N_PAD, c), jnp.float32)
    hext = hext.at[:, :d].set(h * kmask[:, None]).at[:, d].set(kmask)
    out = _spmm(a0, hext)
    agg = out[:, :d] * kmask[:, None]
    cnt = (out[:, d] * kmask)[:, None]
    return agg, cnt


def _segment(msg, dst, n):
    return jax.ops.segment_sum(msg, dst, num_segments=n)


def _graph_conv(h, a0, kmask, pp, mean, act):
    agg, cnt = _message_pass(a0, h, kmask)
    return _conv_transform(agg, cnt, h, pp['W_rel'], pp['b_rel'][None, :],
                           pp['W_root'], mean, act)


def _mean_pool(h, ohm):
    ones = jnp.ones((_N_PAD, 1), jnp.float32)
    pad = jnp.zeros((_N_PAD, 128 - _HID - 1), jnp.float32)
    hext = jnp.concatenate([h, ones, pad], axis=1)
    out = _global_pool(ohm, hext)
    s = out[:, :_HID]
    cnt = jnp.maximum(out[:, _HID:_HID + 1], 1.0)
    return s / cnt


def _pad_pool_params(pp):
    w_rel = jnp.zeros((_HID, 128), jnp.float32).at[:, :1].set(pp['W_rel'])
    w_root = jnp.zeros((_HID, 128), jnp.float32).at[:, :1].set(pp['W_root'])
    b = jnp.zeros((1, 128), jnp.float32).at[0, :1].set(pp['b_rel'])
    return w_rel, b, w_root


def _sag_pool(h, a0, node_valid, batch, starts, pp):
    w_rel, b, w_root = _pad_pool_params(pp)
    agg, _ = _message_pass(a0, h, node_valid)
    cnt = jnp.zeros((_N_PAD, 1), jnp.float32)
    score = _conv_transform(agg, cnt, h, w_rel, b, w_root, False, "none")[:, 0]

    n = _N_PAD
    nvalid = _segment(node_valid.astype(jnp.int32), batch, _N_GRAPHS)
    k = jnp.asarray(_KTAB, jnp.int32)[nvalid]
    sort_key = jnp.where(node_valid > 0, -score, jnp.inf)
    order = jnp.lexsort((sort_key, batch))
    rank_sorted = jnp.arange(n, dtype=jnp.int32) - starts[batch[order]]
    rank = jnp.zeros((n,), jnp.int32).at[order].set(rank_sorted)
    keep = (node_valid > 0) & (rank < k[batch])
    keep_f = keep.astype(jnp.float32)
    new_h = jnp.where(keep[:, None], h * jnp.tanh(score)[:, None], 0.0)
    return new_h, keep_f


def kernel(x, edge_index, batch, params):
    src = edge_index[0]
    dst = edge_index[1]
    n_pad_rows = _N_PAD - _N_NODES

    xp = jnp.pad(x, ((0, n_pad_rows), (0, 0)))
    batch_p = jnp.concatenate(
        [batch, jnp.full((n_pad_rows,), _N_GRAPHS - 1, jnp.int32)])
    node_valid = jnp.concatenate(
        [jnp.ones((_N_NODES,), jnp.float32),
         jnp.zeros((n_pad_rows,), jnp.float32)])

    a0 = jnp.zeros((_N_PAD, _N_PAD), jnp.float32).at[dst, src].add(1.0)

    ones_i = jnp.ones((_N_PAD,), jnp.int32)
    total = _segment(ones_i, batch_p, _N_GRAPHS)
    starts = jnp.cumsum(total) - total

    oh = (batch_p[:, None] == jnp.arange(_N_GRAPHS)[None, :]).astype(
        jnp.float32)

    n_layers = 6
    h = _graph_conv(xp, a0, node_valid, params['conv1'], True, "relu")
    ohm = oh * node_valid[:, None]
    xs = [_mean_pool(h, ohm)]
    pool_i = 0
    n_convs = n_layers - 1
    for i in range(n_convs):
        h = _graph_conv(h, a0, node_valid, params['convs'][i], True, "relu")
        xs.append(_mean_pool(h, ohm))
        if i % 2 == 0 and i < n_convs - 1:
            h, node_valid = _sag_pool(
                h, a0, node_valid, batch_p, starts, params['pools'][pool_i])
            ohm = oh * node_valid[:, None]
            pool_i += 1

    z = jnp.concatenate(xs, axis=1)
    n_classes = params['lin2_W'].shape[1]
    w2p = jnp.zeros((_HID, 128), jnp.float32).at[:, :n_classes].set(
        params['lin2_W'])
    b2p = jnp.zeros((1, 128), jnp.float32).at[0, :n_classes].set(
        params['lin2_b'])
    out = _mlp_head(z, params['lin1_W'], params['lin1_b'][None, :],
                    w2p, b2p, n_classes)
    return out[:, :n_classes]
